# baseline (speedup 1.0000x reference)
import functools

import jax
import jax.numpy as jnp
from jax import lax
from jax.experimental import pallas as pl
from jax.experimental.pallas import tpu as pltpu
from jax.experimental.pallas import tpu_sc as plsc

NUM_FACTORS = 32
NUM_ROWS = 1000000
BATCH = 16384
NC = 2
NS = 16
NW = NC * NS
B_PER_W = BATCH // NW
CHUNK = 128
NCHUNK = B_PER_W // CHUNK
SUP_ROWS = NUM_ROWS * NUM_FACTORS // 128

TCHUNK = 512
NCHUNKS_EVEN = 999424 // TCHUNK
CHUNKS_PER_SUB = NCHUNKS_EVEN // NS
TAIL0 = 999424
TAIL1 = 999936


UNROLL = 4


def _transpose_chunk(src_ref, dst_ref, out_hbm, col0, ncols, lanes):
    @plsc.parallel_loop(0, ncols // 16, unroll=32)
    def tgroup(g):
        j0 = g * 16
        flat_base = (j0 + lanes) * NUM_FACTORS
        for f in range(NUM_FACTORS):
            v = src_ref[f, pl.ds(j0, 16)]
            plsc.store_scatter(dst_ref, [flat_base + f], v)
    el_off = pl.multiple_of(col0 * NUM_FACTORS, TCHUNK * NUM_FACTORS)
    pltpu.sync_copy(dst_ref.at[pl.ds(0, ncols * NUM_FACTORS)],
                    out_hbm.at[pl.ds(el_off, ncols * NUM_FACTORS)])


def _tr_kernel(ufT_hbm, ifT_hbm, tru_hbm, tri_hbm, in_v, tail_v, out_v, isem):
    c = lax.axis_index("c")
    s = lax.axis_index("s")
    lanes = lax.iota(jnp.int32, 16)

    def do_table(src_hbm, dst_hbm):
        def chunk_body(k, carry):
            col0 = pl.multiple_of((s * CHUNKS_PER_SUB + k) * TCHUNK, TCHUNK)
            for a in range(4):
                pltpu.async_copy(
                    src_hbm.at[pl.ds(a * 8, 8), pl.ds(col0, TCHUNK)],
                    in_v.at[pl.ds(a * 8, 8)], isem)
            pltpu.make_async_copy(src_hbm.at[:, pl.ds(0, TCHUNK)],
                                  in_v, isem).wait()
            _transpose_chunk(in_v, out_v, dst_hbm, col0, TCHUNK, lanes)
            return carry
        lax.fori_loop(0, CHUNKS_PER_SUB, chunk_body, 0)

        @pl.when(s == NS - 1)
        def _tail():
            pltpu.sync_copy(src_hbm.at[:, pl.ds(TAIL0, TCHUNK)], in_v)
            _transpose_chunk(in_v, out_v, dst_hbm, TAIL0, TCHUNK, lanes)
            pltpu.sync_copy(src_hbm.at[:, pl.ds(TAIL1, 64)], tail_v)
            _transpose_chunk(tail_v, out_v, dst_hbm, TAIL1, 64, lanes)

    @pl.when(c == 0)
    def _user():
        do_table(ufT_hbm, tru_hbm)

    @pl.when(c == 1)
    def _item():
        do_table(ifT_hbm, tri_hbm)


def _gather_kernel(user_hbm, item_hbm, tru_hbm, tri_hbm, p_hbm, out_hbm,
                   uidx_v, iidx_v, usup_v, isup_v, uvals_v, ivals_v, p_v,
                   outb_v, sem):
    wid = lax.axis_index("s") * NC + lax.axis_index("c")

    pltpu.sync_copy(user_hbm.at[wid], uidx_v)
    pltpu.sync_copy(item_hbm.at[wid], iidx_v)
    pltpu.sync_copy(p_hbm, p_v)

    for cc in range(NCHUNK):
        for g in range(CHUNK // 16):
            usup_v[cc, pl.ds(g * 16, 16)] = (
                uidx_v[cc, pl.ds(g * 16, 16)] >> 2)
            isup_v[cc, pl.ds(g * 16, 16)] = (
                iidx_v[cc, pl.ds(g * 16, 16)] >> 2)

    w_lo = p_v[pl.ds(0, 16)]
    w_hi = p_v[pl.ds(16, 16)]
    tail = p_v[pl.ds(32, 16)]
    ws = [w_lo[f] for f in range(16)] + [w_hi[f] for f in range(16)]
    bias = tail[0]
    lanes = lax.iota(jnp.int32, 16)

    def round_body(cc, carry):
        ucp = pltpu.async_copy(tru_hbm.at[usup_v.at[cc]], uvals_v, sem)
        icp = pltpu.async_copy(tri_hbm.at[isup_v.at[cc]], ivals_v, sem)
        ucp.wait()
        icp.wait()

        def group(g, inner):
            uiv = uidx_v[cc, pl.ds(g * 16, 16)]
            iiv = iidx_v[cc, pl.ds(g * 16, 16)]
            ucol = (uiv & 3) * NUM_FACTORS
            icol = (iiv & 3) * NUM_FACTORS
            rows = g * 16 + lanes
            acc = jnp.full((16,), bias, dtype=jnp.float32)
            for f in range(NUM_FACTORS):
                uv = plsc.load_gather(uvals_v, [rows, ucol + f])
                iv = plsc.load_gather(ivals_v, [rows, icol + f])
                acc = acc + (uv * iv) * ws[f]
            outb_v[cc, pl.ds(g * 16, 16)] = acc
            return inner

        lax.fori_loop(0, CHUNK // 16, group, 0)
        return carry

    lax.fori_loop(0, NCHUNK, round_body, 0)

    pltpu.sync_copy(outb_v, out_hbm.at[wid])


@jax.jit
def _cf_call(user, item, user_factors, item_factors, fc_w, fc_b):
    mesh = plsc.VectorSubcoreMesh(core_axis_name="c", subcore_axis_name="s")
    sc_params = pltpu.CompilerParams(needs_layout_passes=False)

    tr = functools.partial(
        pl.kernel,
        out_type=(jax.ShapeDtypeStruct((NUM_ROWS * NUM_FACTORS,), jnp.float32),
                  jax.ShapeDtypeStruct((NUM_ROWS * NUM_FACTORS,), jnp.float32)),
        mesh=mesh,
        scratch_types=[
            pltpu.VMEM((NUM_FACTORS, TCHUNK), jnp.float32),
            pltpu.VMEM((NUM_FACTORS, 64), jnp.float32),
            pltpu.VMEM((TCHUNK * NUM_FACTORS,), jnp.float32),
            pltpu.SemaphoreType.DMA,
        ],
        compiler_params=sc_params,
    )(_tr_kernel)

    gt = functools.partial(
        pl.kernel,
        out_type=jax.ShapeDtypeStruct((NW, NCHUNK, CHUNK), jnp.float32),
        mesh=mesh,
        scratch_types=[
            pltpu.VMEM((NCHUNK, CHUNK), jnp.int32),
            pltpu.VMEM((NCHUNK, CHUNK), jnp.int32),
            pltpu.VMEM((NCHUNK, CHUNK), jnp.int32),
            pltpu.VMEM((NCHUNK, CHUNK), jnp.int32),
            pltpu.VMEM((CHUNK, 128), jnp.float32),
            pltpu.VMEM((CHUNK, 128), jnp.float32),
            pltpu.VMEM((48,), jnp.float32),
            pltpu.VMEM((NCHUNK, CHUNK), jnp.float32),
            pltpu.SemaphoreType.DMA,
        ],
        compiler_params=sc_params,
    )(_gather_kernel)

    tru, tri = tr(user_factors.T, item_factors.T)
    tru = tru.reshape(SUP_ROWS, 128)
    tri = tri.reshape(SUP_ROWS, 128)
    user3d = user.reshape(NW, NCHUNK, CHUNK)
    item3d = item.reshape(NW, NCHUNK, CHUNK)
    params = jnp.concatenate(
        [fc_w.reshape(NUM_FACTORS), fc_b, jnp.zeros((15,), jnp.float32)])
    return gt(user3d, item3d, tru, tri, params)


def kernel(user, item, user_factors, item_factors, fc_w, fc_b):
    out = _cf_call(user.astype(jnp.int32), item.astype(jnp.int32),
                   user_factors, item_factors, fc_w, fc_b)
    return out.reshape(BATCH, 1)


# speedup vs baseline: 3.1049x; 3.1049x over previous
import functools

import jax
import jax.numpy as jnp
from jax import lax
from jax.experimental import pallas as pl
from jax.experimental.pallas import tpu as pltpu
from jax.experimental.pallas import tpu_sc as plsc

NUM_FACTORS = 32
NUM_ROWS = 1000000
BATCH = 16384
NC = 2
NS = 16
NW = NC * NS
B_PER_W = BATCH // NW
CHUNK = 128
NCHUNK = B_PER_W // CHUNK
SUP_ROWS = NUM_ROWS * NUM_FACTORS // 128

TCHUNK = 512
NCHUNKS_EVEN = 999424 // TCHUNK
CHUNKS_PER_SUB = NCHUNKS_EVEN // NS
TAIL0 = 999424
TAIL1 = 999936


UNROLL = 4


def _transpose_chunk(src_ref, dst_ref, out_hbm, col0, ncols, lanes):
    @plsc.parallel_loop(0, ncols // 16, unroll=UNROLL)
    def tgroup(g):
        j0 = g * 16
        jv = j0 + lanes
        jv32 = jv * NUM_FACTORS
        for f0 in range(NUM_FACTORS):
            p = (f0 + lanes) & (NUM_FACTORS - 1)
            v = plsc.load_gather(src_ref, [p, jv])
            plsc.store_scatter(dst_ref, [jv32 + p], v)
    el_off = pl.multiple_of(col0 * NUM_FACTORS, TCHUNK * NUM_FACTORS)
    pltpu.sync_copy(dst_ref.at[pl.ds(0, ncols * NUM_FACTORS)],
                    out_hbm.at[pl.ds(el_off, ncols * NUM_FACTORS)])


def _tr_kernel(ufT_hbm, ifT_hbm, tru_hbm, tri_hbm, in_v, tail_v, out_v, isem):
    c = lax.axis_index("c")
    s = lax.axis_index("s")
    lanes = lax.iota(jnp.int32, 16)

    def do_table(src_hbm, dst_hbm):
        def chunk_body(k, carry):
            col0 = pl.multiple_of((s * CHUNKS_PER_SUB + k) * TCHUNK, TCHUNK)
            for a in range(4):
                pltpu.async_copy(
                    src_hbm.at[pl.ds(a * 8, 8), pl.ds(col0, TCHUNK)],
                    in_v.at[pl.ds(a * 8, 8)], isem)
            pltpu.make_async_copy(src_hbm.at[:, pl.ds(0, TCHUNK)],
                                  in_v, isem).wait()
            _transpose_chunk(in_v, out_v, dst_hbm, col0, TCHUNK, lanes)
            return carry
        lax.fori_loop(0, CHUNKS_PER_SUB, chunk_body, 0)

        @pl.when(s == NS - 1)
        def _tail():
            pltpu.sync_copy(src_hbm.at[:, pl.ds(TAIL0, TCHUNK)], in_v)
            _transpose_chunk(in_v, out_v, dst_hbm, TAIL0, TCHUNK, lanes)
            pltpu.sync_copy(src_hbm.at[:, pl.ds(TAIL1, 64)], tail_v)
            _transpose_chunk(tail_v, out_v, dst_hbm, TAIL1, 64, lanes)

    @pl.when(c == 0)
    def _user():
        do_table(ufT_hbm, tru_hbm)

    @pl.when(c == 1)
    def _item():
        do_table(ifT_hbm, tri_hbm)


def _gather_kernel(user_hbm, item_hbm, tru_hbm, tri_hbm, p_hbm, out_hbm,
                   uidx_v, iidx_v, usup_v, isup_v, uvals_v, ivals_v, p_v,
                   outb_v, sem):
    wid = lax.axis_index("s") * NC + lax.axis_index("c")

    pltpu.sync_copy(user_hbm.at[wid], uidx_v)
    pltpu.sync_copy(item_hbm.at[wid], iidx_v)
    pltpu.sync_copy(p_hbm, p_v)

    for cc in range(NCHUNK):
        for g in range(CHUNK // 16):
            usup_v[cc, pl.ds(g * 16, 16)] = (
                uidx_v[cc, pl.ds(g * 16, 16)] >> 2)
            isup_v[cc, pl.ds(g * 16, 16)] = (
                iidx_v[cc, pl.ds(g * 16, 16)] >> 2)

    w_lo = p_v[pl.ds(0, 16)]
    w_hi = p_v[pl.ds(16, 16)]
    tail = p_v[pl.ds(32, 16)]
    ws = [w_lo[f] for f in range(16)] + [w_hi[f] for f in range(16)]
    bias = tail[0]
    lanes = lax.iota(jnp.int32, 16)

    def round_body(cc, carry):
        ucp = pltpu.async_copy(tru_hbm.at[usup_v.at[cc]], uvals_v, sem)
        icp = pltpu.async_copy(tri_hbm.at[isup_v.at[cc]], ivals_v, sem)
        ucp.wait()
        icp.wait()

        def group(g, inner):
            uiv = uidx_v[cc, pl.ds(g * 16, 16)]
            iiv = iidx_v[cc, pl.ds(g * 16, 16)]
            ucol = (uiv & 3) * NUM_FACTORS
            icol = (iiv & 3) * NUM_FACTORS
            rows = g * 16 + lanes
            acc = jnp.full((16,), bias, dtype=jnp.float32)
            for f in range(NUM_FACTORS):
                uv = plsc.load_gather(uvals_v, [rows, ucol + f])
                iv = plsc.load_gather(ivals_v, [rows, icol + f])
                acc = acc + (uv * iv) * ws[f]
            outb_v[cc, pl.ds(g * 16, 16)] = acc
            return inner

        lax.fori_loop(0, CHUNK // 16, group, 0)
        return carry

    lax.fori_loop(0, NCHUNK, round_body, 0)

    pltpu.sync_copy(outb_v, out_hbm.at[wid])


@jax.jit
def _cf_call(user, item, user_factors, item_factors, fc_w, fc_b):
    mesh = plsc.VectorSubcoreMesh(core_axis_name="c", subcore_axis_name="s")
    sc_params = pltpu.CompilerParams(needs_layout_passes=False)

    tr = functools.partial(
        pl.kernel,
        out_type=(jax.ShapeDtypeStruct((NUM_ROWS * NUM_FACTORS,), jnp.float32),
                  jax.ShapeDtypeStruct((NUM_ROWS * NUM_FACTORS,), jnp.float32)),
        mesh=mesh,
        scratch_types=[
            pltpu.VMEM((NUM_FACTORS, TCHUNK), jnp.float32),
            pltpu.VMEM((NUM_FACTORS, 64), jnp.float32),
            pltpu.VMEM((TCHUNK * NUM_FACTORS,), jnp.float32),
            pltpu.SemaphoreType.DMA,
        ],
        compiler_params=sc_params,
    )(_tr_kernel)

    gt = functools.partial(
        pl.kernel,
        out_type=jax.ShapeDtypeStruct((NW, NCHUNK, CHUNK), jnp.float32),
        mesh=mesh,
        scratch_types=[
            pltpu.VMEM((NCHUNK, CHUNK), jnp.int32),
            pltpu.VMEM((NCHUNK, CHUNK), jnp.int32),
            pltpu.VMEM((NCHUNK, CHUNK), jnp.int32),
            pltpu.VMEM((NCHUNK, CHUNK), jnp.int32),
            pltpu.VMEM((CHUNK, 128), jnp.float32),
            pltpu.VMEM((CHUNK, 128), jnp.float32),
            pltpu.VMEM((48,), jnp.float32),
            pltpu.VMEM((NCHUNK, CHUNK), jnp.float32),
            pltpu.SemaphoreType.DMA,
        ],
        compiler_params=sc_params,
    )(_gather_kernel)

    tru, tri = tr(user_factors.T, item_factors.T)
    tru = tru.reshape(SUP_ROWS, 128)
    tri = tri.reshape(SUP_ROWS, 128)
    user3d = user.reshape(NW, NCHUNK, CHUNK)
    item3d = item.reshape(NW, NCHUNK, CHUNK)
    params = jnp.concatenate(
        [fc_w.reshape(NUM_FACTORS), fc_b, jnp.zeros((15,), jnp.float32)])
    return gt(user3d, item3d, tru, tri, params)


def kernel(user, item, user_factors, item_factors, fc_w, fc_b):
    out = _cf_call(user.astype(jnp.int32), item.astype(jnp.int32),
                   user_factors, item_factors, fc_w, fc_b)
    return out.reshape(BATCH, 1)


# speedup vs baseline: 5.1350x; 1.6538x over previous
import functools

import jax
import jax.numpy as jnp
from jax import lax
from jax.experimental import pallas as pl
from jax.experimental.pallas import tpu as pltpu
from jax.experimental.pallas import tpu_sc as plsc

NUM_FACTORS = 32
NUM_ROWS = 1000000
BATCH = 16384
NC = 2
NS = 16
NW = NC * NS
B_PER_W = BATCH // NW
CHUNK = 128
NCHUNK = B_PER_W // CHUNK
SUP_ROWS = NUM_ROWS * NUM_FACTORS // 128

TCHUNK = 512
NCHUNKS_EVEN = 999424 // TCHUNK
CHUNKS_PER_SUB = NCHUNKS_EVEN // NS
TAIL0 = 999424
TAIL1 = 999936


UNROLL = 4


def _diag_transpose(src_ref, dst_ref, ncols, lanes):
    @plsc.parallel_loop(0, ncols // 16, unroll=UNROLL)
    def tgroup(g):
        j0 = g * 16
        jv = j0 + lanes
        jv32 = jv * NUM_FACTORS
        for f0 in range(NUM_FACTORS):
            p = (f0 + lanes) & (NUM_FACTORS - 1)
            v = plsc.load_gather(src_ref, [p, jv])
            plsc.store_scatter(dst_ref, [jv32 + p], v)


def _tr_kernel(ufT_hbm, ifT_hbm, tru_hbm, tri_hbm, in0_v, in1_v, tail_v,
               out0_v, out1_v, isem, osem):
    c = lax.axis_index("c")
    s = lax.axis_index("s")
    lanes = lax.iota(jnp.int32, 16)
    NCOLS_OUT = TCHUNK * NUM_FACTORS

    def do_table(src_hbm, dst_hbm):
        ins = (in0_v, in1_v)
        outs = (out0_v, out1_v)

        def fire_in(k, p):
            col0 = pl.multiple_of((s * CHUNKS_PER_SUB + k) * TCHUNK, TCHUNK)
            for a in range(4):
                pltpu.async_copy(
                    src_hbm.at[pl.ds(a * 8, 8), pl.ds(col0, TCHUNK)],
                    ins[p].at[pl.ds(a * 8, 8)], isem)

        def wait_in(p):
            pltpu.make_async_copy(src_hbm.at[:, pl.ds(0, TCHUNK)],
                                  ins[p], isem).wait()

        def fire_out(k, p):
            el_off = pl.multiple_of(
                (s * CHUNKS_PER_SUB + k) * NCOLS_OUT, NCOLS_OUT)
            pltpu.async_copy(outs[p],
                             dst_hbm.at[pl.ds(el_off, NCOLS_OUT)], osem)

        def wait_out(p):
            pltpu.make_async_copy(outs[p],
                                  dst_hbm.at[pl.ds(0, NCOLS_OUT)], osem).wait()

        fire_in(0, 0)

        def chunk_body(kk, carry):
            for p in range(2):
                k = kk * 2 + p
                wait_in(p)

                @pl.when(k + 1 < CHUNKS_PER_SUB)
                def _prefetch():
                    fire_in(k + 1, 1 - p)

                @pl.when(k >= 2)
                def _free_out():
                    wait_out(p)

                _diag_transpose(ins[p], outs[p], TCHUNK, lanes)
                fire_out(k, p)
            return carry

        lax.fori_loop(0, CHUNKS_PER_SUB // 2, chunk_body, 0)
        wait_out(0)
        wait_out(1)

        @pl.when(s == NS - 1)
        def _tail():
            pltpu.sync_copy(src_hbm.at[:, pl.ds(TAIL0, TCHUNK)], in0_v)
            _diag_transpose(in0_v, out0_v, TCHUNK, lanes)
            pltpu.sync_copy(out0_v,
                            dst_hbm.at[pl.ds(TAIL0 * NUM_FACTORS, NCOLS_OUT)])
            pltpu.sync_copy(src_hbm.at[:, pl.ds(TAIL1, 64)], tail_v)
            _diag_transpose(tail_v, out0_v, 64, lanes)
            pltpu.sync_copy(
                out0_v.at[pl.ds(0, 64 * NUM_FACTORS)],
                dst_hbm.at[pl.ds(TAIL1 * NUM_FACTORS, 64 * NUM_FACTORS)])

    @pl.when(c == 0)
    def _user():
        do_table(ufT_hbm, tru_hbm)

    @pl.when(c == 1)
    def _item():
        do_table(ifT_hbm, tri_hbm)


def _gather_kernel(user_hbm, item_hbm, tru_hbm, tri_hbm, p_hbm, out_hbm,
                   uidx_v, iidx_v, usup_v, isup_v, uvals_v, ivals_v, p_v,
                   outb_v, sem):
    wid = lax.axis_index("s") * NC + lax.axis_index("c")

    pltpu.sync_copy(user_hbm.at[wid], uidx_v)
    pltpu.sync_copy(item_hbm.at[wid], iidx_v)
    pltpu.sync_copy(p_hbm, p_v)

    for cc in range(NCHUNK):
        for g in range(CHUNK // 16):
            usup_v[cc, pl.ds(g * 16, 16)] = (
                uidx_v[cc, pl.ds(g * 16, 16)] >> 2)
            isup_v[cc, pl.ds(g * 16, 16)] = (
                iidx_v[cc, pl.ds(g * 16, 16)] >> 2)

    w_lo = p_v[pl.ds(0, 16)]
    w_hi = p_v[pl.ds(16, 16)]
    tail = p_v[pl.ds(32, 16)]
    ws = [w_lo[f] for f in range(16)] + [w_hi[f] for f in range(16)]
    bias = tail[0]
    lanes = lax.iota(jnp.int32, 16)

    def round_body(cc, carry):
        ucp = pltpu.async_copy(tru_hbm.at[usup_v.at[cc]], uvals_v, sem)
        icp = pltpu.async_copy(tri_hbm.at[isup_v.at[cc]], ivals_v, sem)
        ucp.wait()
        icp.wait()

        def group(g, inner):
            uiv = uidx_v[cc, pl.ds(g * 16, 16)]
            iiv = iidx_v[cc, pl.ds(g * 16, 16)]
            ucol = (uiv & 3) * NUM_FACTORS
            icol = (iiv & 3) * NUM_FACTORS
            rows = g * 16 + lanes
            acc = jnp.full((16,), bias, dtype=jnp.float32)
            for f in range(NUM_FACTORS):
                uv = plsc.load_gather(uvals_v, [rows, ucol + f])
                iv = plsc.load_gather(ivals_v, [rows, icol + f])
                acc = acc + (uv * iv) * ws[f]
            outb_v[cc, pl.ds(g * 16, 16)] = acc
            return inner

        lax.fori_loop(0, CHUNK // 16, group, 0)
        return carry

    lax.fori_loop(0, NCHUNK, round_body, 0)

    pltpu.sync_copy(outb_v, out_hbm.at[wid])


@jax.jit
def _cf_call(user, item, user_factors, item_factors, fc_w, fc_b):
    mesh = plsc.VectorSubcoreMesh(core_axis_name="c", subcore_axis_name="s")
    sc_params = pltpu.CompilerParams(needs_layout_passes=False)

    tr = functools.partial(
        pl.kernel,
        out_type=(jax.ShapeDtypeStruct((NUM_ROWS * NUM_FACTORS,), jnp.float32),
                  jax.ShapeDtypeStruct((NUM_ROWS * NUM_FACTORS,), jnp.float32)),
        mesh=mesh,
        scratch_types=[
            pltpu.VMEM((NUM_FACTORS, TCHUNK), jnp.float32),
            pltpu.VMEM((NUM_FACTORS, TCHUNK), jnp.float32),
            pltpu.VMEM((NUM_FACTORS, 64), jnp.float32),
            pltpu.VMEM((TCHUNK * NUM_FACTORS,), jnp.float32),
            pltpu.VMEM((TCHUNK * NUM_FACTORS,), jnp.float32),
            pltpu.SemaphoreType.DMA,
            pltpu.SemaphoreType.DMA,
        ],
        compiler_params=sc_params,
    )(_tr_kernel)

    gt = functools.partial(
        pl.kernel,
        out_type=jax.ShapeDtypeStruct((NW, NCHUNK, CHUNK), jnp.float32),
        mesh=mesh,
        scratch_types=[
            pltpu.VMEM((NCHUNK, CHUNK), jnp.int32),
            pltpu.VMEM((NCHUNK, CHUNK), jnp.int32),
            pltpu.VMEM((NCHUNK, CHUNK), jnp.int32),
            pltpu.VMEM((NCHUNK, CHUNK), jnp.int32),
            pltpu.VMEM((CHUNK, 128), jnp.float32),
            pltpu.VMEM((CHUNK, 128), jnp.float32),
            pltpu.VMEM((48,), jnp.float32),
            pltpu.VMEM((NCHUNK, CHUNK), jnp.float32),
            pltpu.SemaphoreType.DMA,
        ],
        compiler_params=sc_params,
    )(_gather_kernel)

    tru, tri = tr(user_factors.T, item_factors.T)
    tru = tru.reshape(SUP_ROWS, 128)
    tri = tri.reshape(SUP_ROWS, 128)
    user3d = user.reshape(NW, NCHUNK, CHUNK)
    item3d = item.reshape(NW, NCHUNK, CHUNK)
    params = jnp.concatenate(
        [fc_w.reshape(NUM_FACTORS), fc_b, jnp.zeros((15,), jnp.float32)])
    return gt(user3d, item3d, tru, tri, params)


def kernel(user, item, user_factors, item_factors, fc_w, fc_b):
    out = _cf_call(user.astype(jnp.int32), item.astype(jnp.int32),
                   user_factors, item_factors, fc_w, fc_b)
    return out.reshape(BATCH, 1)
